# Initial kernel scaffold; baseline (speedup 1.0000x reference)
import jax
import jax.numpy as jnp
from jax.experimental import pallas as pl


def kernel(input_ids, tok_table, pos_table):
    raise NotImplementedError("write your pallas kernel here")


# speedup vs baseline: 1.2610x; 1.2610x over previous
import functools

import jax
import jax.numpy as jnp
from jax import lax
from jax.experimental import pallas as pl
from jax.experimental.pallas import tpu as pltpu
from jax.experimental.pallas import tpu_sc as plsc

_VOCAB = 100000
_H = 128
_MAX_LEN = 2048
_B = 4
_S = 2048

_NC = 2
_NS = 16
_NW = _NC * _NS
_ROWS_PER_W = (_B * _S) // _NW
_IDX_CHUNK = 128
_NCHUNK = _ROWS_PER_W // _IDX_CHUNK
_LANES = 16
_VECS_PER_ROW = _H // _LANES


def _tok_pos_embed_sc(ids_hbm, tok_hbm, pos_hbm, out_hbm,
                      idx_v, tok_v, pos_v, sem):
  wid = lax.axis_index("s") * _NC + lax.axis_index("c")
  base = wid * _ROWS_PER_W
  pos_base = lax.rem(base, _S)

  for j in range(_NCHUNK):
    pltpu.sync_copy(
        ids_hbm.at[pl.ds(base + j * _IDX_CHUNK, _IDX_CHUNK)],
        idx_v.at[j],
    )

  copies = [
      pltpu.async_copy(
          tok_hbm.at[idx_v.at[j]],
          tok_v.at[pl.ds(j * _IDX_CHUNK, _IDX_CHUNK)],
          sem,
      )
      for j in range(_NCHUNK)
  ]
  pltpu.sync_copy(pos_hbm.at[pl.ds(pos_base, _ROWS_PER_W)], pos_v)
  for c in copies:
    c.wait()

  def row_body(r, carry):
    for j in range(_VECS_PER_ROW):
      sl = pl.ds(j * _LANES, _LANES)
      tok_v[r, sl] = tok_v[r, sl] + pos_v[r, sl]
    return carry

  lax.fori_loop(0, _ROWS_PER_W, row_body, 0)

  pltpu.sync_copy(tok_v, out_hbm.at[pl.ds(base, _ROWS_PER_W)])


def kernel(input_ids, tok_table, pos_table):
  b, s = input_ids.shape
  ids_flat = input_ids.reshape(b * s).astype(jnp.int32)

  mesh = plsc.VectorSubcoreMesh(
      core_axis_name="c", subcore_axis_name="s",
      num_cores=_NC, num_subcores=_NS,
  )
  run = pl.kernel(
      _tok_pos_embed_sc,
      out_type=jax.ShapeDtypeStruct((b * s, _H), jnp.float32),
      mesh=mesh,
      scratch_types=[
          pltpu.VMEM((_NCHUNK, _IDX_CHUNK), jnp.int32),
          pltpu.VMEM((_ROWS_PER_W, _H), jnp.float32),
          pltpu.VMEM((_ROWS_PER_W, _H), jnp.float32),
          pltpu.SemaphoreType.DMA,
      ],
  )
  out = run(ids_flat, tok_table, pos_table)
  return out.reshape(b, s, _H)


# speedup vs baseline: 1.2915x; 1.0241x over previous
import jax
import jax.numpy as jnp
from jax import lax
from jax.experimental import pallas as pl
from jax.experimental.pallas import tpu as pltpu
from jax.experimental.pallas import tpu_sc as plsc

_H = 128
_B = 4
_S = 2048

_NC = 2
_NS = 16
_NW = _NC * _NS
_POS_PER_W = _S // _NW
_LANES = 16
_VECS_PER_ROW = _H // _LANES


def _tok_pos_embed_sc(ids_hbm, tok_hbm, pos_hbm, out_hbm,
                      idx_v, tok_v, pos_v, sem_a, sem_b, sem_out):
  wid = lax.axis_index("s") * _NC + lax.axis_index("c")
  pos_base = wid * _POS_PER_W

  for b in range(_B):
    pltpu.sync_copy(ids_hbm.at[b, pl.ds(pos_base, _POS_PER_W)], idx_v.at[b])

  gather_sems = [sem_a, sem_b]

  def gather(b):
    return pltpu.async_copy(
        tok_hbm.at[idx_v.at[b]],
        tok_v.at[pl.ds(b * _POS_PER_W, _POS_PER_W)],
        gather_sems[b % 2],
    )

  copies = [gather(0)]
  pltpu.sync_copy(pos_hbm.at[pl.ds(pos_base, _POS_PER_W)], pos_v)

  stores = []
  for b in range(_B):
    if b + 1 < _B:
      copies.append(gather(b + 1))
    copies[b].wait()

    def row_body(r, carry, b=b):
      row = b * _POS_PER_W + r
      for j in range(_VECS_PER_ROW):
        sl = pl.ds(j * _LANES, _LANES)
        plsc.addupdate(tok_v.at[row, sl], pos_v[r, sl])
      return carry

    lax.fori_loop(0, _POS_PER_W, row_body, 0)

    stores.append(pltpu.async_copy(
        tok_v.at[pl.ds(b * _POS_PER_W, _POS_PER_W)],
        out_hbm.at[b, pl.ds(pos_base, _POS_PER_W)],
        sem_out,
    ))

  for s in stores:
    s.wait()


def kernel(input_ids, tok_table, pos_table):
  b, s = input_ids.shape
  if input_ids.dtype != jnp.int32:
    input_ids = input_ids.astype(jnp.int32)

  mesh = plsc.VectorSubcoreMesh(
      core_axis_name="c", subcore_axis_name="s",
      num_cores=_NC, num_subcores=_NS,
  )
  run = pl.kernel(
      _tok_pos_embed_sc,
      out_type=jax.ShapeDtypeStruct((b, s, _H), jnp.float32),
      mesh=mesh,
      scratch_types=[
          pltpu.VMEM((_B, _POS_PER_W), jnp.int32),
          pltpu.VMEM((_B * _POS_PER_W, _H), jnp.float32),
          pltpu.VMEM((_POS_PER_W, _H), jnp.float32),
          pltpu.SemaphoreType.DMA,
          pltpu.SemaphoreType.DMA,
          pltpu.SemaphoreType.DMA,
      ],
  )
  return run(input_ids, tok_table, pos_table)
